# Initial kernel scaffold; baseline (speedup 1.0000x reference)
import jax
import jax.numpy as jnp
from jax.experimental import pallas as pl


def kernel(species, distances, switch, edge_src, edge_dst, angles, ang_distances, ang_switch, ang_edge_dst, angle_src, angle_dst, central_atom):
    raise NotImplementedError("write your pallas kernel here")


# speedup vs baseline: 20.7079x; 20.7079x over previous
import jax
import jax.numpy as jnp
from jax import lax
from jax.experimental import pallas as pl
from jax.experimental.pallas import tpu as pltpu
from jax.experimental.pallas import tpu_sc as plsc

N = 10000
E = 320000
AE = 80000
NA = 160000
ZMAX = 10
NBR = 8
NBA = 6
CUTOFF = 5.0
ANG_CUTOFF = 3.5

NC = 2
NS = 16
L = 16

NH = N // 2 * ZMAX
STRIPE = 3136
HTP = NS * STRIPE
TRASH = NH
CH = 1000
NGRP = (CH + L - 1) // L
NSTR = 8
STR = CH // NSTR

F32 = jnp.float32
I32 = jnp.int32


def _stage1_body(d_ref, sw_ref, ad_ref, asw_ref, th_ref, rt_ref, f2_ref, cs_ref):
    lane_r = lax.broadcasted_iota(I32, rt_ref.shape, 1)
    lf = lane_r.astype(F32)
    cen = lf * (CUTOFF / (NBR - 1))
    sig = CUTOFF / NBR
    v = jnp.exp(-(((d_ref[:] - cen) / sig) ** 2)) * sw_ref[:]
    rt_ref[:] = jnp.where(lane_r < NBR, v, 0.0)

    lane_a = lax.broadcasted_iota(I32, f2_ref.shape, 1)
    lfa = lane_a.astype(F32)
    cen2 = lfa * (ANG_CUTOFF / (NBA - 1))
    sig2 = ANG_CUTOFF / NBA
    v2 = jnp.exp(-(((ad_ref[:] - cen2) / sig2) ** 2)) * asw_ref[:]
    f2_ref[:] = jnp.where(lane_a < NBA, v2, 0.0)

    lane_c = lax.broadcasted_iota(I32, cs_ref.shape, 1)
    nf = (lane_c % 5).astype(F32)
    vc = jnp.cos(nf * th_ref[:])
    cs_ref[:] = jnp.where(lane_c < 15, vc, 0.0)


def _stage1(distances, switch, ang_distances, ang_switch, angles):
    g = 80
    br, ba, bn = E // g, AE // g, NA // g
    return pl.pallas_call(
        _stage1_body,
        grid=(g,),
        in_specs=[
            pl.BlockSpec((br, 1), lambda i: (i, 0)),
            pl.BlockSpec((br, 1), lambda i: (i, 0)),
            pl.BlockSpec((ba, 1), lambda i: (i, 0)),
            pl.BlockSpec((ba, 1), lambda i: (i, 0)),
            pl.BlockSpec((bn, 1), lambda i: (i, 0)),
        ],
        out_specs=[
            pl.BlockSpec((br, L), lambda i: (i, 0)),
            pl.BlockSpec((ba, L), lambda i: (i, 0)),
            pl.BlockSpec((bn, L), lambda i: (i, 0)),
        ],
        out_shape=[
            jax.ShapeDtypeStruct((E, L), F32),
            jax.ShapeDtypeStruct((AE, L), F32),
            jax.ShapeDtypeStruct((NA, L), F32),
        ],
    )(
        distances.reshape(E, 1), switch.reshape(E, 1),
        ang_distances.reshape(AE, 1), ang_switch.reshape(AE, 1),
        angles.reshape(NA, 1),
    )


def _sc_body(species_h, esrc_h, edst_h, rt_h, f2_h, aedst_h, asrc_h, adst_h,
             cent_h, cos_h,
             rad_o, ang_o, ext_o,
             species_v, zb_v, stage_v, extd_v, cos_v,
             s_v, d_v, c_v, sadj_v, dadj_v, idx_v, table):
    cid = lax.axis_index("c")
    sid = lax.axis_index("s")
    iota = lax.iota(I32, L)

    pltpu.sync_copy(species_h, species_v)

    def zb_zero(i, _):
        zb_v[i] = jnp.zeros((L,), F32)
        return _
    lax.fori_loop(0, 196, zb_zero, None)

    def zero_table():
        for t in range(16):
            pltpu.sync_copy(zb_v, table.at[pl.ds(sid * STRIPE + t * 196, 196)])

    def dump_table(out_ref, h):
        pltpu.sync_copy(
            table.at[pl.ds(sid * STRIPE, STRIPE)],
            out_ref.at[pl.ds(cid * (2 * HTP) + h * HTP + sid * STRIPE, STRIPE)])

    zero_table()
    plsc.subcore_barrier()

    def radial_chunk_for(h):
        lo = h * NH

        def radial_chunk(j, _):
            base = cid * (E // NC) + sid * (E // NC // NS) + j * CH
            pltpu.sync_copy(esrc_h.at[pl.ds(base, CH)], s_v.at[pl.ds(0, CH)])
            pltpu.sync_copy(edst_h.at[pl.ds(base, CH)], d_v.at[pl.ds(0, CH)])
            pltpu.sync_copy(rt_h.at[pl.ds(base, CH)], stage_v.at[pl.ds(0, CH)])

            def grp(k, _):
                pos = k * L + iota
                m = pos < CH
                src16 = s_v[pl.ds(k * L, L)]
                dst16 = d_v[pl.ds(k * L, L)]
                z16 = plsc.load_gather(species_v, [dst16], mask=m)
                gidx = src16 * ZMAX + z16 - lo
                ok = (gidx >= 0) & (gidx < NH)
                idx16 = jnp.where(ok, gidx, TRASH)
                plsc.store_scatter(idx_v, [pos // STR, pos % STR], idx16,
                                   mask=m)
                return _
            lax.fori_loop(0, NGRP, grp, None)

            for mm in range(NSTR):
                pltpu.sync_copy(stage_v.at[pl.ds(mm * STR, STR)],
                                table.at[idx_v.at[mm]], add=True)
            return _
        return radial_chunk

    for h in range(2):
        lax.fori_loop(0, E // NC // NS // CH, radial_chunk_for(h), None)
        plsc.subcore_barrier()
        dump_table(rad_o, h)
        plsc.subcore_barrier()
        zero_table()
        plsc.subcore_barrier()

    patb = iota // 5 + 3 * cid

    def ext_chunk(j, _):
        base = sid * (AE // NS) + j * CH
        pltpu.sync_copy(aedst_h.at[pl.ds(base, CH)], d_v.at[pl.ds(0, CH)])
        pltpu.sync_copy(f2_h.at[pl.ds(base, CH)], stage_v.at[pl.ds(0, CH)])

        def row(a, _):
            tiled = plsc.load_gather(stage_v, [jnp.full((L,), a, I32), patb])
            extd_v[a] = tiled
            return _
        lax.fori_loop(0, CH, row, None)

        def grp(k, _):
            pos = k * L + iota
            m = pos < CH
            dd16 = d_v[pl.ds(k * L, L)]
            z16 = plsc.load_gather(species_v, [dd16], mask=m)
            plsc.store_scatter(extd_v, [pos, jnp.full((L,), 15, I32)],
                               z16.astype(F32), mask=m)
            return _
        lax.fori_loop(0, NGRP, grp, None)

        pltpu.sync_copy(extd_v.at[pl.ds(0, CH)],
                        ext_o.at[pl.ds(cid * AE + base, CH)])
        return _
    lax.fori_loop(0, AE // NS // CH, ext_chunk, None)

    plsc.subcore_barrier()

    def ang_chunk_for(h):
        lo = h * NH

        def ang_chunk(j, _):
            base = sid * (NA // NS) + j * CH
            pltpu.sync_copy(asrc_h.at[pl.ds(base, CH)], s_v.at[pl.ds(0, CH)])
            pltpu.sync_copy(adst_h.at[pl.ds(base, CH)], d_v.at[pl.ds(0, CH)])
            pltpu.sync_copy(cent_h.at[pl.ds(base, CH)], c_v.at[pl.ds(0, CH)])
            pltpu.sync_copy(cos_h.at[pl.ds(base, CH)], cos_v.at[pl.ds(0, CH)])

            def grp1(k, _):
                pos = k * L + iota
                m = pos < CH
                s16 = s_v[pl.ds(k * L, L)] + cid * AE
                d16 = d_v[pl.ds(k * L, L)] + cid * AE
                plsc.store_scatter(sadj_v, [pos // STR, pos % STR], s16,
                                   mask=m)
                plsc.store_scatter(dadj_v, [pos // STR, pos % STR], d16,
                                   mask=m)
                return _
            lax.fori_loop(0, NGRP, grp1, None)

            for mm in range(NSTR):
                pltpu.sync_copy(ext_o.at[sadj_v.at[mm]],
                                stage_v.at[pl.ds(mm * STR, STR)])
                pltpu.sync_copy(ext_o.at[dadj_v.at[mm]],
                                extd_v.at[pl.ds(mm * STR, STR)])

            def grp2(k, _):
                pos = k * L + iota
                m = pos < CH
                lane15 = jnp.full((L,), 15, I32)
                zs = plsc.load_gather(stage_v, [pos, lane15], mask=m)
                zd = plsc.load_gather(extd_v, [pos, lane15], mask=m)
                c16 = c_v[pl.ds(k * L, L)]
                zi = zs.astype(I32)
                gidx = c16 * ZMAX + zi - lo
                ok = (zs == zd) & (gidx >= 0) & (gidx < NH)
                idx16 = jnp.where(ok, gidx, TRASH)
                plsc.store_scatter(idx_v, [pos // STR, pos % STR], idx16,
                                   mask=m)
                return _
            lax.fori_loop(0, NGRP, grp2, None)

            def data(a, _):
                stage_v[a] = stage_v[a] * extd_v[a] * cos_v[a]
                return _
            lax.fori_loop(0, CH, data, None)

            for mm in range(NSTR):
                pltpu.sync_copy(stage_v.at[pl.ds(mm * STR, STR)],
                                table.at[idx_v.at[mm]], add=True)
            return _
        return ang_chunk

    for h in range(2):
        lax.fori_loop(0, NA // NS // CH, ang_chunk_for(h), None)
        plsc.subcore_barrier()
        dump_table(ang_o, h)
        if h == 0:
            plsc.subcore_barrier()
            zero_table()
            plsc.subcore_barrier()


def _sc_main(species, edge_src, edge_dst, rt16, f2pad, ang_edge_dst,
             angle_src, angle_dst, central_atom, cos16):
    mesh = plsc.VectorSubcoreMesh(core_axis_name="c", subcore_axis_name="s")
    fn = pl.kernel(
        _sc_body,
        out_type=[
            jax.ShapeDtypeStruct((NC * 2 * HTP, L), F32),
            jax.ShapeDtypeStruct((NC * 2 * HTP, L), F32),
            jax.ShapeDtypeStruct((NC * AE, L), F32),
        ],
        mesh=mesh,
        compiler_params=pltpu.CompilerParams(needs_layout_passes=False,
                                             use_tc_tiling_on_sc=False),
        scratch_types=[
            pltpu.VMEM((N,), I32),
            pltpu.VMEM((196, L), F32),
            pltpu.VMEM((CH + L, L), F32),
            pltpu.VMEM((CH + L, L), F32),
            pltpu.VMEM((CH + L, L), F32),
            pltpu.VMEM((CH + L,), I32),
            pltpu.VMEM((CH + L,), I32),
            pltpu.VMEM((CH + L,), I32),
            pltpu.VMEM((NSTR, STR), I32),
            pltpu.VMEM((NSTR, STR), I32),
            pltpu.VMEM((NSTR, STR), I32),
            pltpu.VMEM_SHARED((HTP, L), F32),
        ],
    )
    return fn(species, edge_src, edge_dst, rt16, f2pad, ang_edge_dst,
              angle_src, angle_dst, central_atom, cos16)


def _stage3_body(sp_ref, r0_ref, r1_ref, oh_ref, rs_ref):
    lane = lax.broadcasted_iota(I32, oh_ref.shape, 1)
    oh_ref[:] = (lane == sp_ref[:]).astype(F32)
    rs_ref[:] = r0_ref[:] + r1_ref[:]


def _stage3(species2, r0, r1):
    g = 10
    nt = N * ZMAX
    return pl.pallas_call(
        _stage3_body,
        grid=(g,),
        in_specs=[
            pl.BlockSpec((N // g, 1), lambda i: (i, 0)),
            pl.BlockSpec((nt // g, L), lambda i: (i, 0)),
            pl.BlockSpec((nt // g, L), lambda i: (i, 0)),
        ],
        out_specs=[
            pl.BlockSpec((N // g, L), lambda i: (i, 0)),
            pl.BlockSpec((nt // g, L), lambda i: (i, 0)),
        ],
        out_shape=[
            jax.ShapeDtypeStruct((N, L), F32),
            jax.ShapeDtypeStruct((nt, L), F32),
        ],
    )(species2, r0, r1)


def _halves(tab, cid):
    b = cid * 2 * HTP
    return jnp.concatenate([tab[b:b + NH], tab[b + HTP:b + HTP + NH]])


def kernel(species, distances, switch, edge_src, edge_dst, angles,
           ang_distances, ang_switch, ang_edge_dst, angle_src, angle_dst,
           central_atom):
    rt16, f2pad, cos16 = _stage1(distances, switch, ang_distances,
                                 ang_switch, angles)
    rad2, ang2, _ext = _sc_main(species, edge_src, edge_dst, rt16, f2pad,
                                ang_edge_dst, angle_src, angle_dst,
                                central_atom, cos16)
    oh16, radsum = _stage3(species.reshape(N, 1), _halves(rad2, 0),
                           _halves(rad2, 1))

    onehot = oh16[:, :ZMAX]
    radial = (radsum.reshape(N, ZMAX, L)[:, :, :NBR]
              .transpose(0, 2, 1).reshape(N, NBR * ZMAX))
    angt = jnp.stack([_halves(ang2, 0), _halves(ang2, 1)])
    angular = (angt.reshape(NC, N, ZMAX, L)[:, :, :, :15]
               .reshape(NC, N, ZMAX, 3, 5)
               .transpose(1, 0, 3, 2, 4).reshape(N, ZMAX * NBA * 5))
    return jnp.concatenate([onehot, radial, angular], axis=1)


# speedup vs baseline: 22.9911x; 1.1103x over previous
import jax
import jax.numpy as jnp
from jax import lax
from jax.experimental import pallas as pl
from jax.experimental.pallas import tpu as pltpu
from jax.experimental.pallas import tpu_sc as plsc

N = 10000
E = 320000
AE = 80000
NA = 160000
ZMAX = 10
NBR = 8
NBA = 6
CUTOFF = 5.0
ANG_CUTOFF = 3.5

NC = 2
NS = 16
L = 16

NH = N // 2 * ZMAX
STRIPE = 3136
HTP = NS * STRIPE
TRASH = NH
CH = 1000
NGRP = (CH + L - 1) // L
NSTR = 8
STR = CH // NSTR

F32 = jnp.float32
I32 = jnp.int32


def _stage1_body(d_ref, sw_ref, ad_ref, asw_ref, th_ref, rt_ref, f2_ref, cs_ref):
    lane_r = lax.broadcasted_iota(I32, rt_ref.shape, 1)
    lf = lane_r.astype(F32)
    cen = lf * (CUTOFF / (NBR - 1))
    sig = CUTOFF / NBR
    v = jnp.exp(-(((d_ref[:] - cen) / sig) ** 2)) * sw_ref[:]
    rt_ref[:] = jnp.where(lane_r < NBR, v, 0.0)

    lane_a = lax.broadcasted_iota(I32, f2_ref.shape, 1)
    lfa = lane_a.astype(F32)
    cen2 = lfa * (ANG_CUTOFF / (NBA - 1))
    sig2 = ANG_CUTOFF / NBA
    v2 = jnp.exp(-(((ad_ref[:] - cen2) / sig2) ** 2)) * asw_ref[:]
    f2_ref[:] = jnp.where(lane_a < NBA, v2, 0.0)

    lane_c = lax.broadcasted_iota(I32, cs_ref.shape, 1)
    nf = (lane_c % 5).astype(F32)
    vc = jnp.cos(nf * th_ref[:])
    cs_ref[:] = jnp.where(lane_c < 15, vc, 0.0)


def _stage1(distances, switch, ang_distances, ang_switch, angles):
    g = 80
    br, ba, bn = E // g, AE // g, NA // g
    return pl.pallas_call(
        _stage1_body,
        grid=(g,),
        in_specs=[
            pl.BlockSpec((br, 1), lambda i: (i, 0)),
            pl.BlockSpec((br, 1), lambda i: (i, 0)),
            pl.BlockSpec((ba, 1), lambda i: (i, 0)),
            pl.BlockSpec((ba, 1), lambda i: (i, 0)),
            pl.BlockSpec((bn, 1), lambda i: (i, 0)),
        ],
        out_specs=[
            pl.BlockSpec((br, L), lambda i: (i, 0)),
            pl.BlockSpec((ba, L), lambda i: (i, 0)),
            pl.BlockSpec((bn, L), lambda i: (i, 0)),
        ],
        out_shape=[
            jax.ShapeDtypeStruct((E, L), F32),
            jax.ShapeDtypeStruct((AE, L), F32),
            jax.ShapeDtypeStruct((NA, L), F32),
        ],
    )(
        distances.reshape(E, 1), switch.reshape(E, 1),
        ang_distances.reshape(AE, 1), ang_switch.reshape(AE, 1),
        angles.reshape(NA, 1),
    )


def _sc_body(species_h, esrc_h, edst_h, rt_h, f2_h, aedst_h, asrc_h, adst_h,
             cent_h, cos_h,
             rad_o, ang_o, ext_o,
             species_v, zb_v, stage_v, extd_v, cos_v,
             s_v, d_v, c_v, sadj_v, dadj_v, idx_v, table):
    cid = lax.axis_index("c")
    sid = lax.axis_index("s")
    iota = lax.iota(I32, L)

    pltpu.sync_copy(species_h, species_v)

    def zb_zero(i, _):
        zb_v[i] = jnp.zeros((L,), F32)
        return _
    lax.fori_loop(0, 196, zb_zero, None)

    def zero_table():
        for t in range(16):
            pltpu.sync_copy(zb_v, table.at[pl.ds(sid * STRIPE + t * 196, 196)])

    def dump_table(out_ref, h):
        pltpu.sync_copy(
            table.at[pl.ds(sid * STRIPE, STRIPE)],
            out_ref.at[pl.ds(cid * (2 * HTP) + h * HTP + sid * STRIPE, STRIPE)])

    zero_table()
    plsc.subcore_barrier()

    def radial_chunk_for(h):
        lo = h * NH

        def radial_chunk(j, _):
            base = cid * (E // NC) + sid * (E // NC // NS) + j * CH
            pltpu.sync_copy(esrc_h.at[pl.ds(base, CH)], s_v.at[pl.ds(0, CH)])
            pltpu.sync_copy(edst_h.at[pl.ds(base, CH)], d_v.at[pl.ds(0, CH)])
            pltpu.sync_copy(rt_h.at[pl.ds(base, CH)], stage_v.at[pl.ds(0, CH)])

            def grp(k, _):
                pos = k * L + iota
                m = pos < CH
                src16 = s_v[pl.ds(k * L, L)]
                dst16 = d_v[pl.ds(k * L, L)]
                z16 = plsc.load_gather(species_v, [dst16], mask=m)
                gidx = src16 * ZMAX + z16 - lo
                ok = (gidx >= 0) & (gidx < NH)
                idx16 = jnp.where(ok, gidx, TRASH)
                plsc.store_scatter(idx_v, [pos // STR, pos % STR], idx16,
                                   mask=m)
                return _
            lax.fori_loop(0, NGRP, grp, None)

            for mm in range(NSTR):
                pltpu.sync_copy(stage_v.at[pl.ds(mm * STR, STR)],
                                table.at[idx_v.at[mm]], add=True)
            return _
        return radial_chunk

    for h in range(2):
        lax.fori_loop(0, E // NC // NS // CH, radial_chunk_for(h), None)
        plsc.subcore_barrier()
        dump_table(rad_o, h)
        plsc.subcore_barrier()
        zero_table()
        plsc.subcore_barrier()

    patb = iota // 5 + 3 * cid

    def ext_chunk(j, _):
        base = sid * (AE // NS) + j * CH
        pltpu.sync_copy(aedst_h.at[pl.ds(base, CH)], d_v.at[pl.ds(0, CH)])
        pltpu.sync_copy(f2_h.at[pl.ds(base, CH)], stage_v.at[pl.ds(0, CH)])

        def row(a, _):
            tiled = plsc.load_gather(stage_v, [jnp.full((L,), a, I32), patb])
            extd_v[a] = tiled
            return _
        lax.fori_loop(0, CH, row, None)

        def grp(k, _):
            pos = k * L + iota
            m = pos < CH
            dd16 = d_v[pl.ds(k * L, L)]
            z16 = plsc.load_gather(species_v, [dd16], mask=m)
            plsc.store_scatter(extd_v, [pos, jnp.full((L,), 15, I32)],
                               z16.astype(F32), mask=m)
            return _
        lax.fori_loop(0, NGRP, grp, None)

        pltpu.sync_copy(extd_v.at[pl.ds(0, CH)],
                        ext_o.at[pl.ds(cid * AE + base, CH)])
        return _
    lax.fori_loop(0, AE // NS // CH, ext_chunk, None)

    plsc.subcore_barrier()

    def ang_chunk_for(h):
        lo = h * NH

        def ang_chunk(j, _):
            base = sid * (NA // NS) + j * CH
            pltpu.sync_copy(asrc_h.at[pl.ds(base, CH)], s_v.at[pl.ds(0, CH)])
            pltpu.sync_copy(adst_h.at[pl.ds(base, CH)], d_v.at[pl.ds(0, CH)])
            pltpu.sync_copy(cent_h.at[pl.ds(base, CH)], c_v.at[pl.ds(0, CH)])
            pltpu.sync_copy(cos_h.at[pl.ds(base, CH)], cos_v.at[pl.ds(0, CH)])

            def grp1(k, _):
                pos = k * L + iota
                m = pos < CH
                s16 = s_v[pl.ds(k * L, L)] + cid * AE
                d16 = d_v[pl.ds(k * L, L)] + cid * AE
                plsc.store_scatter(sadj_v, [pos // STR, pos % STR], s16,
                                   mask=m)
                plsc.store_scatter(dadj_v, [pos // STR, pos % STR], d16,
                                   mask=m)
                return _
            lax.fori_loop(0, NGRP, grp1, None)

            for mm in range(NSTR):
                pltpu.sync_copy(ext_o.at[sadj_v.at[mm]],
                                stage_v.at[pl.ds(mm * STR, STR)])
                pltpu.sync_copy(ext_o.at[dadj_v.at[mm]],
                                extd_v.at[pl.ds(mm * STR, STR)])

            def grp2(k, _):
                pos = k * L + iota
                m = pos < CH
                lane15 = jnp.full((L,), 15, I32)
                zs = plsc.load_gather(stage_v, [pos, lane15], mask=m)
                zd = plsc.load_gather(extd_v, [pos, lane15], mask=m)
                c16 = c_v[pl.ds(k * L, L)]
                zi = zs.astype(I32)
                gidx = c16 * ZMAX + zi - lo
                ok = (zs == zd) & (gidx >= 0) & (gidx < NH)
                idx16 = jnp.where(ok, gidx, TRASH)
                plsc.store_scatter(idx_v, [pos // STR, pos % STR], idx16,
                                   mask=m)
                return _
            lax.fori_loop(0, NGRP, grp2, None)

            def data(a, _):
                stage_v[a] = stage_v[a] * extd_v[a] * cos_v[a]
                return _
            lax.fori_loop(0, CH, data, None)

            for mm in range(NSTR):
                pltpu.sync_copy(stage_v.at[pl.ds(mm * STR, STR)],
                                table.at[idx_v.at[mm]], add=True)
            return _
        return ang_chunk

    for h in range(2):
        lax.fori_loop(0, NA // NS // CH, ang_chunk_for(h), None)
        plsc.subcore_barrier()
        dump_table(ang_o, h)
        if h == 0:
            plsc.subcore_barrier()
            zero_table()
            plsc.subcore_barrier()


def _sc_main(species, edge_src, edge_dst, rt16, f2pad, ang_edge_dst,
             angle_src, angle_dst, central_atom, cos16):
    mesh = plsc.VectorSubcoreMesh(core_axis_name="c", subcore_axis_name="s")
    fn = pl.kernel(
        _sc_body,
        out_type=[
            jax.ShapeDtypeStruct((NC * 2 * HTP, L), F32),
            jax.ShapeDtypeStruct((NC * 2 * HTP, L), F32),
            jax.ShapeDtypeStruct((NC * AE, L), F32),
        ],
        mesh=mesh,
        compiler_params=pltpu.CompilerParams(needs_layout_passes=False,
                                             use_tc_tiling_on_sc=False),
        scratch_types=[
            pltpu.VMEM((N,), I32),
            pltpu.VMEM((196, L), F32),
            pltpu.VMEM((CH + L, L), F32),
            pltpu.VMEM((CH + L, L), F32),
            pltpu.VMEM((CH + L, L), F32),
            pltpu.VMEM((CH + L,), I32),
            pltpu.VMEM((CH + L,), I32),
            pltpu.VMEM((CH + L,), I32),
            pltpu.VMEM((NSTR, STR), I32),
            pltpu.VMEM((NSTR, STR), I32),
            pltpu.VMEM((NSTR, STR), I32),
            pltpu.VMEM_SHARED((HTP, L), F32),
        ],
    )
    return fn(species, edge_src, edge_dst, rt16, f2pad, ang_edge_dst,
              angle_src, angle_dst, central_atom, cos16)


import numpy as _np

def _perm_consts():
    pr = _np.zeros((ZMAX * L, NBR * ZMAX), _np.float32)
    for z in range(ZMAX):
        for b in range(NBR):
            pr[z * L + b, b * ZMAX + z] = 1.0
    pa = _np.zeros((NC, ZMAX * L, ZMAX * NBA * 5), _np.float32)
    for cc in range(NC):
        for z in range(ZMAX):
            for bb in range(3):
                for n in range(5):
                    pa[cc, z * L + bb * 5 + n,
                       (cc * 3 + bb) * 50 + z * 5 + n] = 1.0
    return pr, pa[0], pa[1]

_PR, _PA0, _PA1 = _perm_consts()


def _stage3_body(sp_ref, r0_ref, r1_ref, a0_ref, a1_ref, pr_ref, pa0_ref,
                 pa1_ref, out_ref):
    lane = lax.broadcasted_iota(I32, (sp_ref.shape[0], ZMAX), 1)
    oh = (lane == sp_ref[:]).astype(F32)
    rad = jnp.dot(r0_ref[:] + r1_ref[:], pr_ref[:],
                  preferred_element_type=F32)
    ang = (jnp.dot(a0_ref[:], pa0_ref[:], preferred_element_type=F32)
           + jnp.dot(a1_ref[:], pa1_ref[:], preferred_element_type=F32))
    out_ref[:] = jnp.concatenate([oh, rad, ang], axis=1)


def _stage3(species2, r0, r1, a0, a1):
    g = 10
    blk = N // g
    nf = ZMAX + NBR * ZMAX + ZMAX * NBA * 5
    return pl.pallas_call(
        _stage3_body,
        grid=(g,),
        in_specs=[
            pl.BlockSpec((blk, 1), lambda i: (i, 0)),
            pl.BlockSpec((blk, ZMAX * L), lambda i: (i, 0)),
            pl.BlockSpec((blk, ZMAX * L), lambda i: (i, 0)),
            pl.BlockSpec((blk, ZMAX * L), lambda i: (i, 0)),
            pl.BlockSpec((blk, ZMAX * L), lambda i: (i, 0)),
            pl.BlockSpec((ZMAX * L, NBR * ZMAX), lambda i: (0, 0)),
            pl.BlockSpec((ZMAX * L, ZMAX * NBA * 5), lambda i: (0, 0)),
            pl.BlockSpec((ZMAX * L, ZMAX * NBA * 5), lambda i: (0, 0)),
        ],
        out_specs=pl.BlockSpec((blk, nf), lambda i: (i, 0)),
        out_shape=jax.ShapeDtypeStruct((N, nf), F32),
    )(species2, r0, r1, a0, a1, jnp.asarray(_PR), jnp.asarray(_PA0),
      jnp.asarray(_PA1))


def _halves(tab, cid):
    b = cid * 2 * HTP
    return jnp.concatenate([tab[b:b + NH], tab[b + HTP:b + HTP + NH]])


def kernel(species, distances, switch, edge_src, edge_dst, angles,
           ang_distances, ang_switch, ang_edge_dst, angle_src, angle_dst,
           central_atom):
    rt16, f2pad, cos16 = _stage1(distances, switch, ang_distances,
                                 ang_switch, angles)
    rad2, ang2, _ext = _sc_main(species, edge_src, edge_dst, rt16, f2pad,
                                ang_edge_dst, angle_src, angle_dst,
                                central_atom, cos16)
    return _stage3(species.reshape(N, 1),
                   _halves(rad2, 0).reshape(N, ZMAX * L),
                   _halves(rad2, 1).reshape(N, ZMAX * L),
                   _halves(ang2, 0).reshape(N, ZMAX * L),
                   _halves(ang2, 1).reshape(N, ZMAX * L))


# speedup vs baseline: 26.6412x; 1.1588x over previous
import numpy as _np

import jax
import jax.numpy as jnp
from jax import lax
from jax.experimental import pallas as pl
from jax.experimental.pallas import tpu as pltpu
from jax.experimental.pallas import tpu_sc as plsc

N = 10000
E = 320000
AE = 80000
NA = 160000
ZMAX = 10
NBR = 8
NBA = 6
CUTOFF = 5.0
ANG_CUTOFF = 3.5

NC = 2
NS = 16
L = 16

NH = N // 2 * ZMAX
STRIPE = 3136
HTP = NS * STRIPE
TRASH = NH
CH = 1000
NGRP = (CH + L - 1) // L
NSTR = 8
STR = CH // NSTR
EPT = AE // NS
ZROW = 1280

F32 = jnp.float32
I32 = jnp.int32


def _stage1_body(th_ref, c1_ref):
    c1_ref[:] = jnp.cos(th_ref[:])


def _stage1(angles):
    g = 80
    bn = NA // g
    return pl.pallas_call(
        _stage1_body,
        grid=(g,),
        in_specs=[pl.BlockSpec((bn, 1), lambda i: (i, 0))],
        out_specs=pl.BlockSpec((bn, 1), lambda i: (i, 0)),
        out_shape=jax.ShapeDtypeStruct((NA, 1), F32),
    )(angles.reshape(NA, 1))


def _sc_body(species_h, esrc_h, edst_h, dist_h, sw_h, aedst_h, ad_h, asw_h,
             asrc_h, adst_h, cent_h, c1_h,
             rad_o, ang_o, ext_h, zpk_h,
             species_v, zb_v, stage_v, extd_v,
             a_v, b_v, c_v, f1_v, f2_v,
             svs_v, svd_v, svi_v, svc_v, wbuf_v,
             zpk_v, sadj_v, dadj_v, idx_v, table):
    cid = lax.axis_index("c")
    sid = lax.axis_index("s")
    iota = lax.iota(I32, L)
    lane5 = iota % 5

    rsig = CUTOFF / NBR
    rcen = iota.astype(F32) * (CUTOFF / (NBR - 1))
    patb = iota // 5 + 3 * cid
    asig = ANG_CUTOFF / NBA
    acen = patb.astype(F32) * (ANG_CUTOFF / (NBA - 1))

    pltpu.sync_copy(species_h, species_v)

    def zb_zero(i, _):
        zb_v[i] = jnp.zeros((L,), F32)
        return _
    lax.fori_loop(0, 98, zb_zero, None)

    def zero_table():
        for t in range(32):
            pltpu.sync_copy(zb_v, table.at[pl.ds(sid * STRIPE + t * 98, 98)])

    def dump_table(out_ref, h):
        pltpu.sync_copy(
            table.at[pl.ds(sid * STRIPE, STRIPE)],
            out_ref.at[pl.ds(cid * (2 * HTP) + h * HTP + sid * STRIPE, STRIPE)])

    zero_table()
    plsc.subcore_barrier()

    def radial_chunk_for(h):
        lo = h * NH

        def radial_chunk(j, _):
            base = cid * (E // NC) + sid * (E // NC // NS) + j * CH
            pltpu.sync_copy(esrc_h.at[pl.ds(base, CH)], a_v.at[pl.ds(0, CH)])
            pltpu.sync_copy(edst_h.at[pl.ds(base, CH)], b_v.at[pl.ds(0, CH)])
            pltpu.sync_copy(dist_h.at[pl.ds(base, CH)], f1_v.at[pl.ds(0, CH)])
            pltpu.sync_copy(sw_h.at[pl.ds(base, CH)], f2_v.at[pl.ds(0, CH)])

            def row(a, _):
                sp = jnp.full((L,), a, I32)
                d = plsc.load_gather(f1_v, [sp])
                sw = plsc.load_gather(f2_v, [sp])
                x = (d - rcen) * (1.0 / rsig)
                stage_v[a] = jnp.exp(-(x * x)) * sw
                return _
            lax.fori_loop(0, CH, row, None)

            def grp(k, _):
                pos = k * L + iota
                m = pos < CH
                src16 = a_v[pl.ds(k * L, L)]
                dst16 = b_v[pl.ds(k * L, L)]
                z16 = plsc.load_gather(species_v, [dst16], mask=m)
                gidx = src16 * ZMAX + z16 - lo
                ok = (gidx >= 0) & (gidx < NH)
                idx16 = jnp.where(ok, gidx, TRASH)
                plsc.store_scatter(idx_v, [pos // STR, pos % STR], idx16,
                                   mask=m)
                return _
            lax.fori_loop(0, NGRP, grp, None)

            for mm in range(NSTR):
                pltpu.sync_copy(stage_v.at[pl.ds(mm * STR, STR)],
                                table.at[idx_v.at[mm]], add=True)
            return _
        return radial_chunk

    for h in range(2):
        lax.fori_loop(0, E // NC // NS // CH, radial_chunk_for(h), None)
        plsc.subcore_barrier()
        dump_table(rad_o, h)
        plsc.subcore_barrier()
        zero_table()
        plsc.subcore_barrier()

    def ext_chunk(j, _):
        base = sid * EPT + j * CH
        pltpu.sync_copy(aedst_h.at[pl.ds(base, CH)], a_v.at[pl.ds(0, CH)])
        pltpu.sync_copy(ad_h.at[pl.ds(base, CH)], f1_v.at[pl.ds(0, CH)])
        pltpu.sync_copy(asw_h.at[pl.ds(base, CH)], f2_v.at[pl.ds(0, CH)])

        def row(a, _):
            sp = jnp.full((L,), a, I32)
            d = plsc.load_gather(f1_v, [sp])
            sw = plsc.load_gather(f2_v, [sp])
            x = (d - acen) * (1.0 / asig)
            extd_v[a] = jnp.exp(-(x * x)) * sw
            return _
        lax.fori_loop(0, CH, row, None)

        def grp(k, _):
            pos = k * L + iota
            m = pos < CH
            dd16 = a_v[pl.ds(k * L, L)]
            z16 = plsc.load_gather(species_v, [dd16], mask=m)
            plsc.store_scatter(extd_v, [pos, jnp.full((L,), 15, I32)],
                               z16.astype(F32), mask=m)
            b_v[pl.ds(k * L, L)] = z16
            return _
        lax.fori_loop(0, NGRP, grp, None)

        def packgrp(k, _):
            w = k * L + iota
            mm2 = w < (CH // 4)
            e0 = w * 4
            g0 = plsc.load_gather(b_v, [e0], mask=mm2)
            g1 = plsc.load_gather(b_v, [e0 + 1], mask=mm2)
            g2 = plsc.load_gather(b_v, [e0 + 2], mask=mm2)
            g3 = plsc.load_gather(b_v, [e0 + 3], mask=mm2)
            pk = g0 + g1 * 256 + g2 * 65536 + g3 * 16777216
            plsc.store_scatter(wbuf_v, [j * (CH // 4) + w], pk, mask=mm2)
            return _
        lax.fori_loop(0, (CH // 4 + L - 1) // L, packgrp, None)

        pltpu.sync_copy(extd_v.at[pl.ds(0, CH)],
                        ext_h.at[pl.ds(cid * AE + base, CH)])
        return _
    lax.fori_loop(0, EPT // CH, ext_chunk, None)

    pltpu.sync_copy(wbuf_v, zpk_h.at[cid * NS + sid])
    plsc.subcore_barrier()
    pltpu.sync_copy(zpk_h.at[pl.ds(cid * NS, NS)], zpk_v)

    def ang_chunk_for(h):
        lo = h * NH

        def ang_chunk(j, _):
            base = sid * (NA // NS) + j * CH
            pltpu.sync_copy(asrc_h.at[pl.ds(base, CH)], a_v.at[pl.ds(0, CH)])
            pltpu.sync_copy(adst_h.at[pl.ds(base, CH)], b_v.at[pl.ds(0, CH)])
            pltpu.sync_copy(cent_h.at[pl.ds(base, CH)], c_v.at[pl.ds(0, CH)])
            pltpu.sync_copy(c1_h.at[pl.ds(base, CH)], f1_v.at[pl.ds(0, CH)])

            def filt(k, off):
                pos = k * L + iota
                m = pos < CH
                s16 = a_v[pl.ds(k * L, L)]
                d16 = b_v[pl.ds(k * L, L)]
                cen16 = c_v[pl.ds(k * L, L)]
                c116 = f1_v[pl.ds(k * L, L)]
                ws = plsc.load_gather(zpk_v, [s16 // EPT, (s16 % EPT) // 4],
                                      mask=m)
                zs = (ws >> ((s16 & 3) * 8)) & 255
                wd = plsc.load_gather(zpk_v, [d16 // EPT, (d16 % EPT) // 4],
                                      mask=m)
                zd = (wd >> ((d16 & 3) * 8)) & 255
                gidx = cen16 * ZMAX + zs - lo
                ok = (zs == zd) & (gidx >= 0) & (gidx < NH) & m
                plsc.store_compressed(svs_v.at[pl.ds(off, L)],
                                      s16 + cid * AE, mask=ok)
                plsc.store_compressed(svd_v.at[pl.ds(off, L)],
                                      d16 + cid * AE, mask=ok)
                plsc.store_compressed(svi_v.at[pl.ds(off, L)], gidx, mask=ok)
                plsc.store_compressed(svc_v.at[pl.ds(off, L)], c116, mask=ok)
                return off + jnp.sum(ok.astype(I32))
            ns = lax.fori_loop(0, NGRP, filt, jnp.int32(0))

            def fill(k, _):
                pos = k * L + iota
                mc = pos < CH
                m = pos < ns
                sa = plsc.load_gather(svs_v, [pos], mask=m)
                da = plsc.load_gather(svd_v, [pos], mask=m)
                gi = plsc.load_gather(svi_v, [pos], mask=m)
                plsc.store_scatter(sadj_v, [pos // STR, pos % STR],
                                   jnp.where(m, sa, 0), mask=mc)
                plsc.store_scatter(dadj_v, [pos // STR, pos % STR],
                                   jnp.where(m, da, 0), mask=mc)
                plsc.store_scatter(idx_v, [pos // STR, pos % STR],
                                   jnp.where(m, gi, TRASH), mask=mc)
                return _
            lax.fori_loop(0, NGRP, fill, None)

            nstream = (ns + STR - 1) // STR
            for mm in range(NSTR):
                @pl.when(mm < nstream)
                def _():
                    pltpu.sync_copy(ext_h.at[sadj_v.at[mm]],
                                    stage_v.at[pl.ds(mm * STR, STR)])
                    pltpu.sync_copy(ext_h.at[dadj_v.at[mm]],
                                    extd_v.at[pl.ds(mm * STR, STR)])

            ones = jnp.ones((L,), F32)
            p1 = lane5 == 1
            p2 = lane5 == 2
            p3 = lane5 == 3
            p4 = lane5 == 4
            p15 = iota == 15

            def data(a, _):
                es = stage_v[a]
                ed = extd_v[a]
                c1s = plsc.load_gather(svc_v, [jnp.full((L,), a, I32)])
                c2s = 2.0 * c1s * c1s - 1.0
                c3s = 2.0 * c1s * c2s - c1s
                c4s = 2.0 * c1s * c3s - c2s
                cv = jnp.where(p1, c1s, ones)
                cv = jnp.where(p2, c2s, cv)
                cv = jnp.where(p3, c3s, cv)
                cv = jnp.where(p4, c4s, cv)
                cv = jnp.where(p15, 0.0, cv)
                stage_v[a] = es * ed * cv
                return _
            lax.fori_loop(0, ns, data, None)

            for mm in range(NSTR):
                @pl.when(mm < nstream)
                def _():
                    pltpu.sync_copy(stage_v.at[pl.ds(mm * STR, STR)],
                                    table.at[idx_v.at[mm]], add=True)
            return _
        return ang_chunk

    for h in range(2):
        lax.fori_loop(0, NA // NS // CH, ang_chunk_for(h), None)
        plsc.subcore_barrier()
        dump_table(ang_o, h)
        if h == 0:
            plsc.subcore_barrier()
            zero_table()
            plsc.subcore_barrier()


def _sc_main(species, edge_src, edge_dst, distances, switch, ang_edge_dst,
             ang_distances, ang_switch, angle_src, angle_dst, central_atom,
             c1):
    mesh = plsc.VectorSubcoreMesh(core_axis_name="c", subcore_axis_name="s")
    fn = pl.kernel(
        _sc_body,
        out_type=[
            jax.ShapeDtypeStruct((NC * 2 * HTP, L), F32),
            jax.ShapeDtypeStruct((NC * 2 * HTP, L), F32),
            jax.ShapeDtypeStruct((NC * AE, L), F32),
            jax.ShapeDtypeStruct((NC * NS, ZROW), I32),
        ],
        mesh=mesh,
        compiler_params=pltpu.CompilerParams(needs_layout_passes=False,
                                             use_tc_tiling_on_sc=False),
        scratch_types=[
            pltpu.VMEM((N,), I32),
            pltpu.VMEM((98, L), F32),
            pltpu.VMEM((CH + L, L), F32),
            pltpu.VMEM((CH + L, L), F32),
            pltpu.VMEM((CH + L,), I32),
            pltpu.VMEM((CH + L,), I32),
            pltpu.VMEM((CH + L,), I32),
            pltpu.VMEM((CH + L,), F32),
            pltpu.VMEM((CH + L,), F32),
            pltpu.VMEM((CH + L,), I32),
            pltpu.VMEM((CH + L,), I32),
            pltpu.VMEM((CH + L,), I32),
            pltpu.VMEM((CH + L,), F32),
            pltpu.VMEM((ZROW,), I32),
            pltpu.VMEM((NS, ZROW), I32),
            pltpu.VMEM((NSTR, STR), I32),
            pltpu.VMEM((NSTR, STR), I32),
            pltpu.VMEM((NSTR, STR), I32),
            pltpu.VMEM_SHARED((HTP, L), F32),
        ],
    )
    return fn(species, edge_src, edge_dst, distances, switch, ang_edge_dst,
              ang_distances, ang_switch, angle_src, angle_dst, central_atom,
              c1)


def _perm_consts():
    pr = _np.zeros((ZMAX * L, NBR * ZMAX), _np.float32)
    for z in range(ZMAX):
        for b in range(NBR):
            pr[z * L + b, b * ZMAX + z] = 1.0
    pa = _np.zeros((NC, ZMAX * L, ZMAX * NBA * 5), _np.float32)
    for cc in range(NC):
        for z in range(ZMAX):
            for bb in range(3):
                for n in range(5):
                    pa[cc, z * L + bb * 5 + n,
                       (cc * 3 + bb) * 50 + z * 5 + n] = 1.0
    return pr, pa[0], pa[1]

_PR, _PA0, _PA1 = _perm_consts()


def _stage3_body(sp_ref, r0_ref, r1_ref, a0_ref, a1_ref, pr_ref, pa0_ref,
                 pa1_ref, out_ref):
    lane = lax.broadcasted_iota(I32, (sp_ref.shape[0], ZMAX), 1)
    oh = (lane == sp_ref[:]).astype(F32)
    rad = jnp.dot(r0_ref[:] + r1_ref[:], pr_ref[:],
                  preferred_element_type=F32)
    ang = (jnp.dot(a0_ref[:], pa0_ref[:], preferred_element_type=F32)
           + jnp.dot(a1_ref[:], pa1_ref[:], preferred_element_type=F32))
    out_ref[:] = jnp.concatenate([oh, rad, ang], axis=1)


def _stage3(species2, r0, r1, a0, a1):
    g = 10
    blk = N // g
    nf = ZMAX + NBR * ZMAX + ZMAX * NBA * 5
    return pl.pallas_call(
        _stage3_body,
        grid=(g,),
        in_specs=[
            pl.BlockSpec((blk, 1), lambda i: (i, 0)),
            pl.BlockSpec((blk, ZMAX * L), lambda i: (i, 0)),
            pl.BlockSpec((blk, ZMAX * L), lambda i: (i, 0)),
            pl.BlockSpec((blk, ZMAX * L), lambda i: (i, 0)),
            pl.BlockSpec((blk, ZMAX * L), lambda i: (i, 0)),
            pl.BlockSpec((ZMAX * L, NBR * ZMAX), lambda i: (0, 0)),
            pl.BlockSpec((ZMAX * L, ZMAX * NBA * 5), lambda i: (0, 0)),
            pl.BlockSpec((ZMAX * L, ZMAX * NBA * 5), lambda i: (0, 0)),
        ],
        out_specs=pl.BlockSpec((blk, nf), lambda i: (i, 0)),
        out_shape=jax.ShapeDtypeStruct((N, nf), F32),
    )(species2, r0, r1, a0, a1, jnp.asarray(_PR), jnp.asarray(_PA0),
      jnp.asarray(_PA1))


def _halves(tab, cid):
    b = cid * 2 * HTP
    return jnp.concatenate([tab[b:b + NH], tab[b + HTP:b + HTP + NH]])


def kernel(species, distances, switch, edge_src, edge_dst, angles,
           ang_distances, ang_switch, ang_edge_dst, angle_src, angle_dst,
           central_atom):
    c1 = _stage1(angles).reshape(NA)
    rad2, ang2, _ext, _zpk = _sc_main(species, edge_src, edge_dst, distances,
                                      switch, ang_edge_dst, ang_distances,
                                      ang_switch, angle_src, angle_dst,
                                      central_atom, c1)
    return _stage3(species.reshape(N, 1),
                   _halves(rad2, 0).reshape(N, ZMAX * L),
                   _halves(rad2, 1).reshape(N, ZMAX * L),
                   _halves(ang2, 0).reshape(N, ZMAX * L),
                   _halves(ang2, 1).reshape(N, ZMAX * L))


# speedup vs baseline: 45.2898x; 1.7000x over previous
import numpy as _np

import jax
import jax.numpy as jnp
from jax import lax
from jax.experimental import pallas as pl
from jax.experimental.pallas import tpu as pltpu
from jax.experimental.pallas import tpu_sc as plsc

N = 10000
E = 320000
AE = 80000
NA = 160000
ZMAX = 10
NBR = 8
NBA = 6
CUTOFF = 5.0
ANG_CUTOFF = 3.5

NC = 2
NS = 16
L = 16

NH = N // 2 * ZMAX
STRIPE = 3136
HTP = NS * STRIPE
TRASH = NH
CH = 1000
NGRP = (CH + L - 1) // L
NSTR = 8
STR = CH // NSTR
EPT = AE // NS
ZROW = 1280

F32 = jnp.float32
I32 = jnp.int32


def _stage1_body(th_ref, c1_ref):
    c1_ref[:] = jnp.cos(th_ref[:])


def _stage1(angles):
    return pl.pallas_call(
        _stage1_body,
        out_shape=jax.ShapeDtypeStruct((NA // 128, 128), F32),
    )(angles.reshape(NA // 128, 128))


def _sc_body(species_h, esrc_h, edst_h, dist_h, sw_h, aedst_h, ad_h, asw_h,
             asrc_h, adst_h, cent_h, c1_h,
             rad_o, ang_o, ext_h, zpk_h,
             species_v, zb_v, stage_v, extd_v,
             a_v, b_v, c_v, f1_v, f2_v,
             svs_v, svd_v, svi_v, svc_v, wbuf_v,
             zpk_v, sadj_v, dadj_v, idx_v, table):
    cid = lax.axis_index("c")
    sid = lax.axis_index("s")
    iota = lax.iota(I32, L)
    lane5 = iota % 5

    rsig = CUTOFF / NBR
    rcen = iota.astype(F32) * (CUTOFF / (NBR - 1))
    patb = iota // 5 + 3 * cid
    asig = ANG_CUTOFF / NBA
    acen = patb.astype(F32) * (ANG_CUTOFF / (NBA - 1))

    pltpu.sync_copy(species_h, species_v)

    def zb_zero(i, _):
        zb_v[i] = jnp.zeros((L,), F32)
        return _
    lax.fori_loop(0, 98, zb_zero, None)

    def zero_table():
        for t in range(32):
            pltpu.sync_copy(zb_v, table.at[pl.ds(sid * STRIPE + t * 98, 98)])

    LAST = NH - 15 * STRIPE

    def dump_table(out_ref, h):
        obase = cid * (2 * NH) + h * NH + sid * STRIPE

        @pl.when(sid < NS - 1)
        def _():
            pltpu.sync_copy(table.at[pl.ds(sid * STRIPE, STRIPE)],
                            out_ref.at[pl.ds(obase, STRIPE)])

        @pl.when(sid == NS - 1)
        def _():
            pltpu.sync_copy(table.at[pl.ds(sid * STRIPE, LAST)],
                            out_ref.at[pl.ds(obase, LAST)])

    zero_table()
    plsc.subcore_barrier()

    def radial_chunk_for(h):
        lo = h * NH

        def radial_chunk(j, _):
            base = cid * (E // NC) + sid * (E // NC // NS) + j * CH
            pltpu.sync_copy(esrc_h.at[pl.ds(base, CH)], a_v.at[pl.ds(0, CH)])
            pltpu.sync_copy(edst_h.at[pl.ds(base, CH)], b_v.at[pl.ds(0, CH)])
            pltpu.sync_copy(dist_h.at[pl.ds(base, CH)], f1_v.at[pl.ds(0, CH)])
            pltpu.sync_copy(sw_h.at[pl.ds(base, CH)], f2_v.at[pl.ds(0, CH)])

            @plsc.parallel_loop(0, CH, unroll=4)
            def row(a):
                sp = jnp.full((L,), a, I32)
                d = plsc.load_gather(f1_v, [sp])
                sw = plsc.load_gather(f2_v, [sp])
                x = (d - rcen) * (1.0 / rsig)
                stage_v[a] = jnp.exp(-(x * x)) * sw

            @plsc.parallel_loop(0, NGRP, unroll=4)
            def grp(k):
                pos = k * L + iota
                m = pos < CH
                src16 = a_v[pl.ds(k * L, L)]
                dst16 = b_v[pl.ds(k * L, L)]
                z16 = plsc.load_gather(species_v, [dst16], mask=m)
                gidx = src16 * ZMAX + z16 - lo
                ok = (gidx >= 0) & (gidx < NH)
                idx16 = jnp.where(ok, gidx, TRASH)
                plsc.store_scatter(idx_v, [pos // STR, pos % STR], idx16,
                                   mask=m)

            for mm in range(NSTR):
                pltpu.sync_copy(stage_v.at[pl.ds(mm * STR, STR)],
                                table.at[idx_v.at[mm]], add=True)
            return _
        return radial_chunk

    for h in range(2):
        lax.fori_loop(0, E // NC // NS // CH, radial_chunk_for(h), None)
        plsc.subcore_barrier()
        dump_table(rad_o, h)
        plsc.subcore_barrier()
        zero_table()
        plsc.subcore_barrier()

    def ext_chunk(j, _):
        base = sid * EPT + j * CH
        pltpu.sync_copy(aedst_h.at[pl.ds(base, CH)], a_v.at[pl.ds(0, CH)])
        pltpu.sync_copy(ad_h.at[pl.ds(base, CH)], f1_v.at[pl.ds(0, CH)])
        pltpu.sync_copy(asw_h.at[pl.ds(base, CH)], f2_v.at[pl.ds(0, CH)])

        @plsc.parallel_loop(0, CH, unroll=4)
        def row(a):
            sp = jnp.full((L,), a, I32)
            d = plsc.load_gather(f1_v, [sp])
            sw = plsc.load_gather(f2_v, [sp])
            x = (d - acen) * (1.0 / asig)
            extd_v[a] = jnp.exp(-(x * x)) * sw

        def grp(k, _):
            pos = k * L + iota
            m = pos < CH
            dd16 = a_v[pl.ds(k * L, L)]
            z16 = plsc.load_gather(species_v, [dd16], mask=m)
            plsc.store_scatter(extd_v, [pos, jnp.full((L,), 15, I32)],
                               z16.astype(F32), mask=m)
            b_v[pl.ds(k * L, L)] = z16
            return _
        lax.fori_loop(0, NGRP, grp, None)

        def packgrp(k, _):
            w = k * L + iota
            mm2 = w < (CH // 4)
            e0 = w * 4
            g0 = plsc.load_gather(b_v, [e0], mask=mm2)
            g1 = plsc.load_gather(b_v, [e0 + 1], mask=mm2)
            g2 = plsc.load_gather(b_v, [e0 + 2], mask=mm2)
            g3 = plsc.load_gather(b_v, [e0 + 3], mask=mm2)
            pk = g0 + g1 * 256 + g2 * 65536 + g3 * 16777216
            plsc.store_scatter(wbuf_v, [j * (CH // 4) + w], pk, mask=mm2)
            return _
        lax.fori_loop(0, (CH // 4 + L - 1) // L, packgrp, None)

        pltpu.sync_copy(extd_v.at[pl.ds(0, CH)],
                        ext_h.at[pl.ds(cid * AE + base, CH)])
        return _
    lax.fori_loop(0, EPT // CH, ext_chunk, None)

    pltpu.sync_copy(wbuf_v, zpk_h.at[cid * NS + sid])
    plsc.subcore_barrier()
    pltpu.sync_copy(zpk_h.at[pl.ds(cid * NS, NS)], zpk_v)

    def ang_chunk_for(h):
        lo = h * NH

        def ang_chunk(j, _):
            base = sid * (NA // NS) + j * CH
            pltpu.sync_copy(asrc_h.at[pl.ds(base, CH)], a_v.at[pl.ds(0, CH)])
            pltpu.sync_copy(adst_h.at[pl.ds(base, CH)], b_v.at[pl.ds(0, CH)])
            pltpu.sync_copy(cent_h.at[pl.ds(base, CH)], c_v.at[pl.ds(0, CH)])
            pltpu.sync_copy(c1_h.at[pl.ds(base, CH)], f1_v.at[pl.ds(0, CH)])

            def filt(k, off):
                pos = k * L + iota
                m = pos < CH
                s16 = a_v[pl.ds(k * L, L)]
                d16 = b_v[pl.ds(k * L, L)]
                cen16 = c_v[pl.ds(k * L, L)]
                c116 = f1_v[pl.ds(k * L, L)]
                ws = plsc.load_gather(zpk_v, [s16 // EPT, (s16 % EPT) // 4],
                                      mask=m)
                zs = (ws >> ((s16 & 3) * 8)) & 255
                wd = plsc.load_gather(zpk_v, [d16 // EPT, (d16 % EPT) // 4],
                                      mask=m)
                zd = (wd >> ((d16 & 3) * 8)) & 255
                gidx = cen16 * ZMAX + zs - lo
                ok = (zs == zd) & (gidx >= 0) & (gidx < NH) & m
                plsc.store_compressed(svs_v.at[pl.ds(off, L)],
                                      s16 + cid * AE, mask=ok)
                plsc.store_compressed(svd_v.at[pl.ds(off, L)],
                                      d16 + cid * AE, mask=ok)
                plsc.store_compressed(svi_v.at[pl.ds(off, L)], gidx, mask=ok)
                plsc.store_compressed(svc_v.at[pl.ds(off, L)], c116, mask=ok)
                return off + jnp.sum(ok.astype(I32))
            ns = lax.fori_loop(0, NGRP, filt, jnp.int32(0))

            @plsc.parallel_loop(0, NGRP, unroll=4)
            def fill(k):
                pos = k * L + iota
                mc = pos < CH
                m = pos < ns
                sa = plsc.load_gather(svs_v, [pos], mask=m)
                da = plsc.load_gather(svd_v, [pos], mask=m)
                gi = plsc.load_gather(svi_v, [pos], mask=m)
                plsc.store_scatter(sadj_v, [pos // STR, pos % STR],
                                   jnp.where(m, sa, 0), mask=mc)
                plsc.store_scatter(dadj_v, [pos // STR, pos % STR],
                                   jnp.where(m, da, 0), mask=mc)
                plsc.store_scatter(idx_v, [pos // STR, pos % STR],
                                   jnp.where(m, gi, TRASH), mask=mc)

            nstream = (ns + STR - 1) // STR
            for mm in range(NSTR):
                @pl.when(mm < nstream)
                def _():
                    pltpu.sync_copy(ext_h.at[sadj_v.at[mm]],
                                    stage_v.at[pl.ds(mm * STR, STR)])
                    pltpu.sync_copy(ext_h.at[dadj_v.at[mm]],
                                    extd_v.at[pl.ds(mm * STR, STR)])

            ones = jnp.ones((L,), F32)
            p1 = lane5 == 1
            p2 = lane5 == 2
            p3 = lane5 == 3
            p4 = lane5 == 4
            p15 = iota == 15

            def data(a, _):
                es = stage_v[a]
                ed = extd_v[a]
                c1s = plsc.load_gather(svc_v, [jnp.full((L,), a, I32)])
                c2s = 2.0 * c1s * c1s - 1.0
                c3s = 2.0 * c1s * c2s - c1s
                c4s = 2.0 * c1s * c3s - c2s
                cv = jnp.where(p1, c1s, ones)
                cv = jnp.where(p2, c2s, cv)
                cv = jnp.where(p3, c3s, cv)
                cv = jnp.where(p4, c4s, cv)
                cv = jnp.where(p15, 0.0, cv)
                stage_v[a] = es * ed * cv
                return _
            lax.fori_loop(0, ns, data, None)

            for mm in range(NSTR):
                @pl.when(mm < nstream)
                def _():
                    pltpu.sync_copy(stage_v.at[pl.ds(mm * STR, STR)],
                                    table.at[idx_v.at[mm]], add=True)
            return _
        return ang_chunk

    for h in range(2):
        lax.fori_loop(0, NA // NS // CH, ang_chunk_for(h), None)
        plsc.subcore_barrier()
        dump_table(ang_o, h)
        if h == 0:
            plsc.subcore_barrier()
            zero_table()
            plsc.subcore_barrier()


def _sc_main(species, edge_src, edge_dst, distances, switch, ang_edge_dst,
             ang_distances, ang_switch, angle_src, angle_dst, central_atom,
             c1):
    mesh = plsc.VectorSubcoreMesh(core_axis_name="c", subcore_axis_name="s")
    fn = pl.kernel(
        _sc_body,
        out_type=[
            jax.ShapeDtypeStruct((NC * 2 * NH, L), F32),
            jax.ShapeDtypeStruct((NC * 2 * NH, L), F32),
            jax.ShapeDtypeStruct((NC * AE, L), F32),
            jax.ShapeDtypeStruct((NC * NS, ZROW), I32),
        ],
        mesh=mesh,
        compiler_params=pltpu.CompilerParams(needs_layout_passes=False,
                                             use_tc_tiling_on_sc=False),
        scratch_types=[
            pltpu.VMEM((N,), I32),
            pltpu.VMEM((98, L), F32),
            pltpu.VMEM((CH + L, L), F32),
            pltpu.VMEM((CH + L, L), F32),
            pltpu.VMEM((CH + L,), I32),
            pltpu.VMEM((CH + L,), I32),
            pltpu.VMEM((CH + L,), I32),
            pltpu.VMEM((CH + L,), F32),
            pltpu.VMEM((CH + L,), F32),
            pltpu.VMEM((CH + L,), I32),
            pltpu.VMEM((CH + L,), I32),
            pltpu.VMEM((CH + L,), I32),
            pltpu.VMEM((CH + L,), F32),
            pltpu.VMEM((ZROW,), I32),
            pltpu.VMEM((NS, ZROW), I32),
            pltpu.VMEM((NSTR, STR), I32),
            pltpu.VMEM((NSTR, STR), I32),
            pltpu.VMEM((NSTR, STR), I32),
            pltpu.VMEM_SHARED((HTP, L), F32),
        ],
    )
    return fn(species, edge_src, edge_dst, distances, switch, ang_edge_dst,
              ang_distances, ang_switch, angle_src, angle_dst, central_atom,
              c1)


def _perm_consts():
    pr = _np.zeros((ZMAX * L, NBR * ZMAX), _np.float32)
    for z in range(ZMAX):
        for b in range(NBR):
            pr[z * L + b, b * ZMAX + z] = 1.0
    pa = _np.zeros((NC, ZMAX * L, ZMAX * NBA * 5), _np.float32)
    for cc in range(NC):
        for z in range(ZMAX):
            for bb in range(3):
                for n in range(5):
                    pa[cc, z * L + bb * 5 + n,
                       (cc * 3 + bb) * 50 + z * 5 + n] = 1.0
    return pr, pa[0], pa[1]

_PR, _PA0, _PA1 = _perm_consts()


def _stage3_body(sp_ref, r0_ref, r1_ref, a0_ref, a1_ref, pr_ref, pa0_ref,
                 pa1_ref, out_ref):
    lane = lax.broadcasted_iota(I32, (sp_ref.shape[0], ZMAX), 1)
    oh = (lane == sp_ref[:]).astype(F32)
    rad = jnp.dot(r0_ref[:] + r1_ref[:], pr_ref[:],
                  preferred_element_type=F32)
    ang = (jnp.dot(a0_ref[:], pa0_ref[:], preferred_element_type=F32)
           + jnp.dot(a1_ref[:], pa1_ref[:], preferred_element_type=F32))
    out_ref[:] = jnp.concatenate([oh, rad, ang], axis=1)


def _stage3(species2, r0, r1, a0, a1):
    g = 10
    blk = N // g
    nf = ZMAX + NBR * ZMAX + ZMAX * NBA * 5
    return pl.pallas_call(
        _stage3_body,
        grid=(g,),
        in_specs=[
            pl.BlockSpec((blk, 1), lambda i: (i, 0)),
            pl.BlockSpec((blk, ZMAX * L), lambda i: (i, 0)),
            pl.BlockSpec((blk, ZMAX * L), lambda i: (i, 0)),
            pl.BlockSpec((blk, ZMAX * L), lambda i: (i, 0)),
            pl.BlockSpec((blk, ZMAX * L), lambda i: (i, 0)),
            pl.BlockSpec((ZMAX * L, NBR * ZMAX), lambda i: (0, 0)),
            pl.BlockSpec((ZMAX * L, ZMAX * NBA * 5), lambda i: (0, 0)),
            pl.BlockSpec((ZMAX * L, ZMAX * NBA * 5), lambda i: (0, 0)),
        ],
        out_specs=pl.BlockSpec((blk, nf), lambda i: (i, 0)),
        out_shape=jax.ShapeDtypeStruct((N, nf), F32),
    )(species2, r0, r1, a0, a1, jnp.asarray(_PR), jnp.asarray(_PA0),
      jnp.asarray(_PA1))


def kernel(species, distances, switch, edge_src, edge_dst, angles,
           ang_distances, ang_switch, ang_edge_dst, angle_src, angle_dst,
           central_atom):
    c1 = _stage1(angles).reshape(NA)
    rad2, ang2, _ext, _zpk = _sc_main(species, edge_src, edge_dst, distances,
                                      switch, ang_edge_dst, ang_distances,
                                      ang_switch, angle_src, angle_dst,
                                      central_atom, c1)
    nh2 = 2 * NH
    return _stage3(species.reshape(N, 1),
                   rad2[:nh2].reshape(N, ZMAX * L),
                   rad2[nh2:].reshape(N, ZMAX * L),
                   ang2[:nh2].reshape(N, ZMAX * L),
                   ang2[nh2:].reshape(N, ZMAX * L))


# speedup vs baseline: 45.9255x; 1.0140x over previous
import numpy as _np

import jax
import jax.numpy as jnp
from jax import lax
from jax.experimental import pallas as pl
from jax.experimental.pallas import tpu as pltpu
from jax.experimental.pallas import tpu_sc as plsc

N = 10000
E = 320000
AE = 80000
NA = 160000
ZMAX = 10
NBR = 8
NBA = 6
CUTOFF = 5.0
ANG_CUTOFF = 3.5

NC = 2
NS = 16
L = 16

NH = N // 2 * ZMAX
STRIPE = 3136
HTP = NS * STRIPE
TRASH = NH
CH = 1000
NGRP = (CH + L - 1) // L
NSTR = 8
STR = CH // NSTR
EPT = AE // NS
ZROW = 1280

F32 = jnp.float32
I32 = jnp.int32


def _stage1_body(th_ref, c1_ref):
    c1_ref[:] = jnp.cos(th_ref[:])


def _stage1(angles):
    return pl.pallas_call(
        _stage1_body,
        out_shape=jax.ShapeDtypeStruct((NA // 128, 128), F32),
    )(angles.reshape(NA // 128, 128))


def _sc_body(species_h, esrc_h, edst_h, dist_h, sw_h, aedst_h, ad_h, asw_h,
             asrc_h, adst_h, cent_h, c1_h,
             rad_o, ang_o, ext_h, zpk_h,
             species_v, zb_v, stage_v, extd_v,
             a_v, b_v, c_v, f1_v, f2_v,
             svs_v, svd_v, svi_v, svc_v, wbuf_v,
             zpk_v, sadj_v, dadj_v, idx_v, table, sem1, sem2):
    cid = lax.axis_index("c")
    sid = lax.axis_index("s")
    iota = lax.iota(I32, L)
    lane5 = iota % 5

    rsig = CUTOFF / NBR
    rcen = iota.astype(F32) * (CUTOFF / (NBR - 1))
    patb = iota // 5 + 3 * cid
    asig = ANG_CUTOFF / NBA
    acen = patb.astype(F32) * (ANG_CUTOFF / (NBA - 1))

    pltpu.sync_copy(species_h, species_v)

    def zb_zero(i, _):
        zb_v[i] = jnp.zeros((L,), F32)
        return _
    lax.fori_loop(0, 98, zb_zero, None)

    def zero_table():
        for t in range(32):
            pltpu.sync_copy(zb_v, table.at[pl.ds(sid * STRIPE + t * 98, 98)])

    LAST = NH - 15 * STRIPE

    def dump_table(out_ref, h):
        obase = cid * (2 * NH) + h * NH + sid * STRIPE

        @pl.when(sid < NS - 1)
        def _():
            pltpu.sync_copy(table.at[pl.ds(sid * STRIPE, STRIPE)],
                            out_ref.at[pl.ds(obase, STRIPE)])

        @pl.when(sid == NS - 1)
        def _():
            pltpu.sync_copy(table.at[pl.ds(sid * STRIPE, LAST)],
                            out_ref.at[pl.ds(obase, LAST)])

    zero_table()
    plsc.subcore_barrier()

    def radial_chunk_for(h):
        lo = h * NH

        def radial_chunk(j, _):
            base = cid * (E // NC) + sid * (E // NC // NS) + j * CH
            cps = [
                pltpu.async_copy(esrc_h.at[pl.ds(base, CH)],
                                 a_v.at[pl.ds(0, CH)], sem1),
                pltpu.async_copy(edst_h.at[pl.ds(base, CH)],
                                 b_v.at[pl.ds(0, CH)], sem1),
                pltpu.async_copy(dist_h.at[pl.ds(base, CH)],
                                 f1_v.at[pl.ds(0, CH)], sem1),
                pltpu.async_copy(sw_h.at[pl.ds(base, CH)],
                                 f2_v.at[pl.ds(0, CH)], sem1),
            ]
            for cp in cps:
                cp.wait()

            @plsc.parallel_loop(0, CH, unroll=4)
            def row(a):
                sp = jnp.full((L,), a, I32)
                d = plsc.load_gather(f1_v, [sp])
                sw = plsc.load_gather(f2_v, [sp])
                x = (d - rcen) * (1.0 / rsig)
                stage_v[a] = jnp.exp(-(x * x)) * sw

            @plsc.parallel_loop(0, NGRP, unroll=4)
            def grp(k):
                pos = k * L + iota
                m = pos < CH
                src16 = a_v[pl.ds(k * L, L)]
                dst16 = b_v[pl.ds(k * L, L)]
                z16 = plsc.load_gather(species_v, [dst16], mask=m)
                gidx = src16 * ZMAX + z16 - lo
                ok = (gidx >= 0) & (gidx < NH)
                idx16 = jnp.where(ok, gidx, TRASH)
                plsc.store_scatter(idx_v, [pos // STR, pos % STR], idx16,
                                   mask=m)

            scs = [pltpu.async_copy(stage_v.at[pl.ds(mm * STR, STR)],
                                    table.at[idx_v.at[mm]], sem2, add=True)
                   for mm in range(NSTR)]
            for cp in scs:
                cp.wait()
            return _
        return radial_chunk

    for h in range(2):
        lax.fori_loop(0, E // NC // NS // CH, radial_chunk_for(h), None)
        plsc.subcore_barrier()
        dump_table(rad_o, h)
        plsc.subcore_barrier()
        zero_table()
        plsc.subcore_barrier()

    def ext_chunk(j, _):
        base = sid * EPT + j * CH
        cps = [
            pltpu.async_copy(aedst_h.at[pl.ds(base, CH)],
                             a_v.at[pl.ds(0, CH)], sem1),
            pltpu.async_copy(ad_h.at[pl.ds(base, CH)],
                             f1_v.at[pl.ds(0, CH)], sem1),
            pltpu.async_copy(asw_h.at[pl.ds(base, CH)],
                             f2_v.at[pl.ds(0, CH)], sem1),
        ]
        for cp in cps:
            cp.wait()

        @plsc.parallel_loop(0, CH, unroll=4)
        def row(a):
            sp = jnp.full((L,), a, I32)
            d = plsc.load_gather(f1_v, [sp])
            sw = plsc.load_gather(f2_v, [sp])
            x = (d - acen) * (1.0 / asig)
            extd_v[a] = jnp.exp(-(x * x)) * sw

        def grp(k, _):
            pos = k * L + iota
            m = pos < CH
            dd16 = a_v[pl.ds(k * L, L)]
            z16 = plsc.load_gather(species_v, [dd16], mask=m)
            plsc.store_scatter(extd_v, [pos, jnp.full((L,), 15, I32)],
                               z16.astype(F32), mask=m)
            b_v[pl.ds(k * L, L)] = z16
            return _
        lax.fori_loop(0, NGRP, grp, None)

        def packgrp(k, _):
            w = k * L + iota
            mm2 = w < (CH // 4)
            e0 = w * 4
            g0 = plsc.load_gather(b_v, [e0], mask=mm2)
            g1 = plsc.load_gather(b_v, [e0 + 1], mask=mm2)
            g2 = plsc.load_gather(b_v, [e0 + 2], mask=mm2)
            g3 = plsc.load_gather(b_v, [e0 + 3], mask=mm2)
            pk = g0 + g1 * 256 + g2 * 65536 + g3 * 16777216
            plsc.store_scatter(wbuf_v, [j * (CH // 4) + w], pk, mask=mm2)
            return _
        lax.fori_loop(0, (CH // 4 + L - 1) // L, packgrp, None)

        pltpu.sync_copy(extd_v.at[pl.ds(0, CH)],
                        ext_h.at[pl.ds(cid * AE + base, CH)])
        return _
    lax.fori_loop(0, EPT // CH, ext_chunk, None)

    pltpu.sync_copy(wbuf_v, zpk_h.at[cid * NS + sid])
    plsc.subcore_barrier()
    pltpu.sync_copy(zpk_h.at[pl.ds(cid * NS, NS)], zpk_v)

    def ang_chunk_for(h):
        lo = h * NH

        def ang_chunk(j, _):
            base = sid * (NA // NS) + j * CH
            cps = [
                pltpu.async_copy(asrc_h.at[pl.ds(base, CH)],
                                 a_v.at[pl.ds(0, CH)], sem1),
                pltpu.async_copy(adst_h.at[pl.ds(base, CH)],
                                 b_v.at[pl.ds(0, CH)], sem1),
                pltpu.async_copy(cent_h.at[pl.ds(base, CH)],
                                 c_v.at[pl.ds(0, CH)], sem1),
                pltpu.async_copy(c1_h.at[pl.ds(base, CH)],
                                 f1_v.at[pl.ds(0, CH)], sem1),
            ]
            for cp in cps:
                cp.wait()

            def filt(k, off):
                pos = k * L + iota
                m = pos < CH
                s16 = a_v[pl.ds(k * L, L)]
                d16 = b_v[pl.ds(k * L, L)]
                cen16 = c_v[pl.ds(k * L, L)]
                c116 = f1_v[pl.ds(k * L, L)]
                ws = plsc.load_gather(zpk_v, [s16 // EPT, (s16 % EPT) // 4],
                                      mask=m)
                zs = (ws >> ((s16 & 3) * 8)) & 255
                wd = plsc.load_gather(zpk_v, [d16 // EPT, (d16 % EPT) // 4],
                                      mask=m)
                zd = (wd >> ((d16 & 3) * 8)) & 255
                gidx = cen16 * ZMAX + zs - lo
                ok = (zs == zd) & (gidx >= 0) & (gidx < NH) & m
                plsc.store_compressed(svs_v.at[pl.ds(off, L)],
                                      s16 + cid * AE, mask=ok)
                plsc.store_compressed(svd_v.at[pl.ds(off, L)],
                                      d16 + cid * AE, mask=ok)
                plsc.store_compressed(svi_v.at[pl.ds(off, L)], gidx, mask=ok)
                plsc.store_compressed(svc_v.at[pl.ds(off, L)], c116, mask=ok)
                return off + jnp.sum(ok.astype(I32))
            ns = lax.fori_loop(0, NGRP, filt, jnp.int32(0))

            @plsc.parallel_loop(0, NGRP, unroll=4)
            def fill(k):
                pos = k * L + iota
                mc = pos < CH
                m = pos < ns
                sa = plsc.load_gather(svs_v, [pos], mask=m)
                da = plsc.load_gather(svd_v, [pos], mask=m)
                gi = plsc.load_gather(svi_v, [pos], mask=m)
                plsc.store_scatter(sadj_v, [pos // STR, pos % STR],
                                   jnp.where(m, sa, 0), mask=mc)
                plsc.store_scatter(dadj_v, [pos // STR, pos % STR],
                                   jnp.where(m, da, 0), mask=mc)
                plsc.store_scatter(idx_v, [pos // STR, pos % STR],
                                   jnp.where(m, gi, TRASH), mask=mc)

            nstream = (ns + STR - 1) // STR
            for mm in range(NSTR):
                @pl.when(mm < nstream)
                def _():
                    pltpu.sync_copy(ext_h.at[sadj_v.at[mm]],
                                    stage_v.at[pl.ds(mm * STR, STR)])
                    pltpu.sync_copy(ext_h.at[dadj_v.at[mm]],
                                    extd_v.at[pl.ds(mm * STR, STR)])

            ones = jnp.ones((L,), F32)
            p1 = lane5 == 1
            p2 = lane5 == 2
            p3 = lane5 == 3
            p4 = lane5 == 4
            p15 = iota == 15

            def data(a, _):
                es = stage_v[a]
                ed = extd_v[a]
                c1s = plsc.load_gather(svc_v, [jnp.full((L,), a, I32)])
                c2s = 2.0 * c1s * c1s - 1.0
                c3s = 2.0 * c1s * c2s - c1s
                c4s = 2.0 * c1s * c3s - c2s
                cv = jnp.where(p1, c1s, ones)
                cv = jnp.where(p2, c2s, cv)
                cv = jnp.where(p3, c3s, cv)
                cv = jnp.where(p4, c4s, cv)
                cv = jnp.where(p15, 0.0, cv)
                stage_v[a] = es * ed * cv
                return _
            lax.fori_loop(0, ns, data, None)

            for mm in range(NSTR):
                @pl.when(mm < nstream)
                def _():
                    pltpu.sync_copy(stage_v.at[pl.ds(mm * STR, STR)],
                                    table.at[idx_v.at[mm]], add=True)
            return _
        return ang_chunk

    for h in range(2):
        lax.fori_loop(0, NA // NS // CH, ang_chunk_for(h), None)
        plsc.subcore_barrier()
        dump_table(ang_o, h)
        if h == 0:
            plsc.subcore_barrier()
            zero_table()
            plsc.subcore_barrier()


def _sc_main(species, edge_src, edge_dst, distances, switch, ang_edge_dst,
             ang_distances, ang_switch, angle_src, angle_dst, central_atom,
             c1):
    mesh = plsc.VectorSubcoreMesh(core_axis_name="c", subcore_axis_name="s")
    fn = pl.kernel(
        _sc_body,
        out_type=[
            jax.ShapeDtypeStruct((NC * 2 * NH, L), F32),
            jax.ShapeDtypeStruct((NC * 2 * NH, L), F32),
            jax.ShapeDtypeStruct((NC * AE, L), F32),
            jax.ShapeDtypeStruct((NC * NS, ZROW), I32),
        ],
        mesh=mesh,
        compiler_params=pltpu.CompilerParams(needs_layout_passes=False,
                                             use_tc_tiling_on_sc=False),
        scratch_types=[
            pltpu.VMEM((N,), I32),
            pltpu.VMEM((98, L), F32),
            pltpu.VMEM((CH + L, L), F32),
            pltpu.VMEM((CH + L, L), F32),
            pltpu.VMEM((CH + L,), I32),
            pltpu.VMEM((CH + L,), I32),
            pltpu.VMEM((CH + L,), I32),
            pltpu.VMEM((CH + L,), F32),
            pltpu.VMEM((CH + L,), F32),
            pltpu.VMEM((CH + L,), I32),
            pltpu.VMEM((CH + L,), I32),
            pltpu.VMEM((CH + L,), I32),
            pltpu.VMEM((CH + L,), F32),
            pltpu.VMEM((ZROW,), I32),
            pltpu.VMEM((NS, ZROW), I32),
            pltpu.VMEM((NSTR, STR), I32),
            pltpu.VMEM((NSTR, STR), I32),
            pltpu.VMEM((NSTR, STR), I32),
            pltpu.VMEM_SHARED((HTP, L), F32),
            pltpu.SemaphoreType.DMA,
            pltpu.SemaphoreType.DMA,
        ],
    )
    return fn(species, edge_src, edge_dst, distances, switch, ang_edge_dst,
              ang_distances, ang_switch, angle_src, angle_dst, central_atom,
              c1)


def _perm_consts():
    pr = _np.zeros((ZMAX * L, NBR * ZMAX), _np.float32)
    for z in range(ZMAX):
        for b in range(NBR):
            pr[z * L + b, b * ZMAX + z] = 1.0
    pa = _np.zeros((NC, ZMAX * L, ZMAX * NBA * 5), _np.float32)
    for cc in range(NC):
        for z in range(ZMAX):
            for bb in range(3):
                for n in range(5):
                    pa[cc, z * L + bb * 5 + n,
                       (cc * 3 + bb) * 50 + z * 5 + n] = 1.0
    return pr, pa[0], pa[1]

_PR, _PA0, _PA1 = _perm_consts()


def _stage3_body(sp_ref, r0_ref, r1_ref, a0_ref, a1_ref, pr_ref, pa0_ref,
                 pa1_ref, out_ref):
    lane = lax.broadcasted_iota(I32, (sp_ref.shape[0], ZMAX), 1)
    oh = (lane == sp_ref[:]).astype(F32)
    rad = jnp.dot(r0_ref[:] + r1_ref[:], pr_ref[:],
                  preferred_element_type=F32)
    ang = (jnp.dot(a0_ref[:], pa0_ref[:], preferred_element_type=F32)
           + jnp.dot(a1_ref[:], pa1_ref[:], preferred_element_type=F32))
    out_ref[:] = jnp.concatenate([oh, rad, ang], axis=1)


def _stage3(species2, r0, r1, a0, a1):
    g = 10
    blk = N // g
    nf = ZMAX + NBR * ZMAX + ZMAX * NBA * 5
    return pl.pallas_call(
        _stage3_body,
        grid=(g,),
        in_specs=[
            pl.BlockSpec((blk, 1), lambda i: (i, 0)),
            pl.BlockSpec((blk, ZMAX * L), lambda i: (i, 0)),
            pl.BlockSpec((blk, ZMAX * L), lambda i: (i, 0)),
            pl.BlockSpec((blk, ZMAX * L), lambda i: (i, 0)),
            pl.BlockSpec((blk, ZMAX * L), lambda i: (i, 0)),
            pl.BlockSpec((ZMAX * L, NBR * ZMAX), lambda i: (0, 0)),
            pl.BlockSpec((ZMAX * L, ZMAX * NBA * 5), lambda i: (0, 0)),
            pl.BlockSpec((ZMAX * L, ZMAX * NBA * 5), lambda i: (0, 0)),
        ],
        out_specs=pl.BlockSpec((blk, nf), lambda i: (i, 0)),
        out_shape=jax.ShapeDtypeStruct((N, nf), F32),
    )(species2, r0, r1, a0, a1, jnp.asarray(_PR), jnp.asarray(_PA0),
      jnp.asarray(_PA1))


def kernel(species, distances, switch, edge_src, edge_dst, angles,
           ang_distances, ang_switch, ang_edge_dst, angle_src, angle_dst,
           central_atom):
    c1 = _stage1(angles).reshape(NA)
    rad2, ang2, _ext, _zpk = _sc_main(species, edge_src, edge_dst, distances,
                                      switch, ang_edge_dst, ang_distances,
                                      ang_switch, angle_src, angle_dst,
                                      central_atom, c1)
    nh2 = 2 * NH
    return _stage3(species.reshape(N, 1),
                   rad2[:nh2].reshape(N, ZMAX * L),
                   rad2[nh2:].reshape(N, ZMAX * L),
                   ang2[:nh2].reshape(N, ZMAX * L),
                   ang2[nh2:].reshape(N, ZMAX * L))
